# baseline (speedup 1.0000x reference)
import functools

import jax
import jax.numpy as jnp
from jax import lax
from jax.experimental import pallas as pl
from jax.experimental.pallas import tpu as pltpu
from jax.experimental.pallas import tpu_sc as plsc

_B, _C, _COUT = 4, 768, 768
_N = 1024
_HE = 256
_KN = 32
_KE = 8
_NW = 32


def _make_sc_gather_sum(num_rows, k_fan, table_rows, feat):
    epw = num_rows // _NW
    m = 32 // k_fan
    G = epw // m
    gpc = 32 // m
    nc = G // gpc
    fw = feat // 32
    mesh = plsc.VectorSubcoreMesh(core_axis_name="c", subcore_axis_name="s",
                                  num_cores=2, num_subcores=16)

    D = 4

    @functools.partial(
        pl.kernel,
        out_type=jax.ShapeDtypeStruct((num_rows, feat), jnp.float32),
        mesh=mesh,
        scratch_types=[
            pltpu.VMEM((G, 32), jnp.int32),
            pltpu.VMEM((32, feat // 2), jnp.int32),
            pltpu.VMEM((32, feat // 2), jnp.int32),
            pltpu.VMEM((32, feat // 2), jnp.int32),
            pltpu.VMEM((32, feat // 2), jnp.int32),
            pltpu.VMEM((32, feat), jnp.float32),
            pltpu.SemaphoreType.DMA,
            pltpu.SemaphoreType.DMA,
            pltpu.SemaphoreType.DMA,
            pltpu.SemaphoreType.DMA,
            pltpu.SemaphoreType.DMA,
        ],
    )
    def sc_kernel(table_hbm, idx_hbm, out_hbm, idx_v,
                  b0, b1, b2, b3, o0, s0, s1, s2, s3, t0):
        bufs, sems = (b0, b1, b2, b3), (s0, s1, s2, s3)
        wid = lax.axis_index("s") * 2 + lax.axis_index("c")
        base = wid * epw
        pltpu.sync_copy(idx_hbm.at[wid], idx_v)
        for g in range(min(D, G)):
            pltpu.async_copy(table_hbm.at[idx_v.at[g]], bufs[g % D], sems[g % D])

        for c in range(nc):
            if c >= 1:
                pltpu.make_async_copy(
                    o0, out_hbm.at[pl.ds(base + (c - 1) * 32, 32)], t0).wait()
            for gg in range(gpc):
                g = c * gpc + gg
                buf, sem = bufs[g % D], sems[g % D]
                pltpu.make_async_copy(table_hbm.at[idx_v.at[g]], buf, sem).wait()

                def reduce_t(t, _, buf=buf, gg=gg):
                    sl = pl.ds(t * 16, 16)
                    for e in range(m):
                        los, his = [], []
                        for j in range(k_fan):
                            w = buf[e * k_fan + j, sl]
                            los.append(lax.bitcast_convert_type(
                                jnp.left_shift(w, 16), jnp.float32))
                            his.append(lax.bitcast_convert_type(
                                jnp.bitwise_and(w, jnp.int32(-65536)),
                                jnp.float32))
                        for vals in (los, his):
                            while len(vals) > 1:
                                nxt = [vals[i] + vals[i + 1]
                                       for i in range(0, len(vals) - 1, 2)]
                                if len(vals) % 2:
                                    nxt.append(vals[-1])
                                vals[:] = nxt
                        row = gg * m + e
                        o0[row, pl.ds(t * 32, 16)] = los[0]
                        o0[row, pl.ds(t * 32 + 16, 16)] = his[0]
                    return _

                lax.fori_loop(0, fw, reduce_t, 0)
                if g + D < G:
                    pltpu.async_copy(table_hbm.at[idx_v.at[g + D]], buf, sem)
            pltpu.async_copy(o0, out_hbm.at[pl.ds(base + c * 32, 32)], t0)

        pltpu.make_async_copy(
            o0, out_hbm.at[pl.ds(base + (nc - 1) * 32, 32)], t0).wait()

    return sc_kernel


_sc_cache = {}


def _sc_gather_sum(num_rows, k_fan, table_rows, feat):
    key = (num_rows, k_fan, table_rows, feat)
    if key not in _sc_cache:
        _sc_cache[key] = _make_sc_gather_sum(num_rows, k_fan, table_rows, feat)
    return _sc_cache[key]


def _pack_rows(rows_bf16):
    u = lax.bitcast_convert_type(rows_bf16, jnp.uint16).astype(jnp.int32)
    return u[:, 0::2] | (u[:, 1::2] << 16)


def _perm():
    return jnp.arange(_C).reshape(_C // 32, 16, 2).transpose(0, 2, 1).reshape(_C)


def _tc1_body(eps_ref, h_ref, c_ref, w_ref, b_ref, o_ref):
    e = lax.dot_general(h_ref[...], w_ref[...], (((1,), (1,)), ((), ())),
                        preferred_element_type=jnp.float32)
    e = jnp.maximum(e + b_ref[...], 0.0)
    o_ref[...] = (e + (1.0 + eps_ref[0]) * c_ref[...]).astype(jnp.bfloat16)


def _tc1(hsum, centers_rows, W1p, b1, eps):
    blk = 256
    return pl.pallas_call(
        _tc1_body,
        grid=(_B * _HE // blk,),
        in_specs=[
            pl.BlockSpec(memory_space=pltpu.SMEM),
            pl.BlockSpec((blk, _C), lambda i: (i, 0)),
            pl.BlockSpec((blk, _C), lambda i: (i, 0)),
            pl.BlockSpec((_C, _C), lambda i: (0, 0)),
            pl.BlockSpec((1, _C), lambda i: (0, 0)),
        ],
        out_specs=pl.BlockSpec((blk, _C), lambda i: (i, 0)),
        out_shape=jax.ShapeDtypeStruct((_B * _HE, _C), jnp.bfloat16),
    )(eps, hsum, centers_rows, W1p, b1.reshape(1, _C))


def _tc2_body(g_ref, w_ref, b_ref, o_ref):
    o = lax.dot_general(w_ref[...], g_ref[0], (((1,), (1,)), ((), ())),
                        preferred_element_type=jnp.float32)
    o_ref[0] = jnp.maximum(o + b_ref[...], 0.0)


def _tc2(gsum, W2p, b2):
    g3 = gsum.reshape(_B, _N, _C)
    return pl.pallas_call(
        _tc2_body,
        grid=(_B,),
        in_specs=[
            pl.BlockSpec((1, _N, _C), lambda b: (b, 0, 0)),
            pl.BlockSpec((_COUT, _C), lambda b: (0, 0)),
            pl.BlockSpec((_COUT, 1), lambda b: (0, 0)),
        ],
        out_specs=pl.BlockSpec((1, _COUT, _N), lambda b: (b, 0, 0)),
        out_shape=jax.ShapeDtypeStruct((_B, _COUT, _N), jnp.float32),
    )(g3, W2p, b2.reshape(_COUT, 1))


def kernel(x, hyperedge_matrix, point_hyperedge_index, centers, W1, b1, W2, b2, eps):
    xT = jnp.transpose(x[..., 0].astype(jnp.bfloat16),
                       (0, 2, 1)).reshape(_B * (_N + 1), _C)
    t1 = _pack_rows(xT)
    centers_rows = jnp.transpose(centers[:, :, :_HE, 0], (0, 2, 1)).reshape(_B * _HE, _C)
    perm = _perm()

    boff_n = (jnp.arange(_B, dtype=jnp.int32) * (_N + 1))[:, None, None]
    idx1 = (hyperedge_matrix.astype(jnp.int32) + boff_n).reshape(_B * _HE, _KN)
    idx1t = idx1.reshape(_NW, -1, 32)

    boff_e = (jnp.arange(_B, dtype=jnp.int32) * _HE)[:, None, None]
    idx2 = (point_hyperedge_index.astype(jnp.int32) + boff_e).reshape(_B * _N, _KE)
    idx2t = idx2.reshape(_NW, -1, 32)

    hsum = _sc_gather_sum(_B * _HE, _KN, _B * (_N + 1), _C)(t1, idx1t)
    e_rows = _tc1(hsum, centers_rows, W1[:, perm], b1, eps)
    t2 = _pack_rows(e_rows)
    gsum = _sc_gather_sum(_B * _N, _KE, _B * _HE, _C)(t2, idx2t)
    return _tc2(gsum, W2[:, perm], b2)


# speedup vs baseline: 3.4088x; 3.4088x over previous
import functools

import jax
import jax.numpy as jnp
from jax import lax
from jax.experimental import pallas as pl
from jax.experimental.pallas import tpu as pltpu
from jax.experimental.pallas import tpu_sc as plsc

_B, _C, _COUT = 4, 768, 768
_N = 1024
_HE = 256
_KN = 32
_KE = 8
_NW = 32


def _make_sc_gather_sum(num_rows, k_fan, table_rows, feat):
    epw = num_rows // _NW
    m = 32 // k_fan
    G = epw // m
    gpc = 32 // m
    nc = G // gpc
    fw = feat // 32
    mesh = plsc.VectorSubcoreMesh(core_axis_name="c", subcore_axis_name="s",
                                  num_cores=2, num_subcores=16)

    D = 4

    @functools.partial(
        pl.kernel,
        out_type=jax.ShapeDtypeStruct((num_rows, feat), jnp.float32),
        mesh=mesh,
        scratch_types=[
            pltpu.VMEM((G, 32), jnp.int32),
            pltpu.VMEM((32, feat // 2), jnp.int32),
            pltpu.VMEM((32, feat // 2), jnp.int32),
            pltpu.VMEM((32, feat // 2), jnp.int32),
            pltpu.VMEM((32, feat // 2), jnp.int32),
            pltpu.VMEM((32, feat), jnp.float32),
            pltpu.SemaphoreType.DMA,
            pltpu.SemaphoreType.DMA,
            pltpu.SemaphoreType.DMA,
            pltpu.SemaphoreType.DMA,
            pltpu.SemaphoreType.DMA,
        ],
    )
    def sc_kernel(table_hbm, idx_hbm, out_hbm, idx_v,
                  b0, b1, b2, b3, o0, s0, s1, s2, s3, t0):
        bufs, sems = (b0, b1, b2, b3), (s0, s1, s2, s3)
        wid = lax.axis_index("s") * 2 + lax.axis_index("c")
        base = wid * epw
        pltpu.sync_copy(idx_hbm.at[wid], idx_v)
        for g in range(min(D, G)):
            pltpu.async_copy(table_hbm.at[idx_v.at[g]], bufs[g % D], sems[g % D])

        for c in range(nc):
            if c >= 1:
                pltpu.make_async_copy(
                    o0, out_hbm.at[pl.ds(base + (c - 1) * 32, 32)], t0).wait()
            for gg in range(gpc):
                g = c * gpc + gg
                buf, sem = bufs[g % D], sems[g % D]
                pltpu.make_async_copy(table_hbm.at[idx_v.at[g]], buf, sem).wait()

                def reduce_t(t, _, buf=buf, gg=gg):
                    sl = pl.ds(t * 16, 16)
                    for e in range(m):
                        los, his = [], []
                        for j in range(k_fan):
                            w = buf[e * k_fan + j, sl]
                            los.append(lax.bitcast_convert_type(
                                jnp.left_shift(w, 16), jnp.float32))
                            his.append(lax.bitcast_convert_type(
                                jnp.bitwise_and(w, jnp.int32(-65536)),
                                jnp.float32))
                        for vals in (los, his):
                            while len(vals) > 1:
                                nxt = [vals[i] + vals[i + 1]
                                       for i in range(0, len(vals) - 1, 2)]
                                if len(vals) % 2:
                                    nxt.append(vals[-1])
                                vals[:] = nxt
                        row = gg * m + e
                        o0[row, pl.ds(t * 32, 16)] = los[0]
                        o0[row, pl.ds(t * 32 + 16, 16)] = his[0]
                    return _

                lax.fori_loop(0, fw, reduce_t, 0)
                if g + D < G:
                    pltpu.async_copy(table_hbm.at[idx_v.at[g + D]], buf, sem)
            pltpu.async_copy(o0, out_hbm.at[pl.ds(base + c * 32, 32)], t0)

        pltpu.make_async_copy(
            o0, out_hbm.at[pl.ds(base + (nc - 1) * 32, 32)], t0).wait()

    return sc_kernel


_sc_cache = {}


def _sc_gather_sum(num_rows, k_fan, table_rows, feat):
    key = (num_rows, k_fan, table_rows, feat)
    if key not in _sc_cache:
        _sc_cache[key] = _make_sc_gather_sum(num_rows, k_fan, table_rows, feat)
    return _sc_cache[key]


def _pack_cmajor(a_f32):
    b, c, v = a_f32.shape
    u = lax.bitcast_convert_type(a_f32.astype(jnp.bfloat16), jnp.uint16)
    r = u.reshape(b, c // 2, 2, v).astype(jnp.int32)
    w = r[:, :, 0, :] | (r[:, :, 1, :] << 16)
    return w.transpose(0, 2, 1).reshape(b * v, c // 2)


def _perm():
    return jnp.arange(_C).reshape(_C // 32, 16, 2).transpose(0, 2, 1).reshape(_C)


def _tc1_body(eps_ref, h_ref, c_ref, w_ref, b_ref, o_ref):
    e = lax.dot_general(w_ref[...], h_ref[...], (((1,), (1,)), ((), ())),
                        preferred_element_type=jnp.float32)
    e = jnp.maximum(e + b_ref[...], 0.0)
    o_ref[0] = e + (1.0 + eps_ref[0]) * c_ref[0]


def _tc1(hsum, centers_cmaj, W1p, b1, eps):
    return pl.pallas_call(
        _tc1_body,
        grid=(_B,),
        in_specs=[
            pl.BlockSpec(memory_space=pltpu.SMEM),
            pl.BlockSpec((_HE, _C), lambda b: (b, 0)),
            pl.BlockSpec((1, _C, _HE), lambda b: (b, 0, 0)),
            pl.BlockSpec((_C, _C), lambda b: (0, 0)),
            pl.BlockSpec((_C, 1), lambda b: (0, 0)),
        ],
        out_specs=pl.BlockSpec((1, _C, _HE), lambda b: (b, 0, 0)),
        out_shape=jax.ShapeDtypeStruct((_B, _C, _HE), jnp.float32),
    )(eps, hsum, centers_cmaj, W1p, b1.reshape(_C, 1))


def _tc2_body(g_ref, w_ref, b_ref, o_ref):
    o = lax.dot_general(w_ref[...], g_ref[0], (((1,), (1,)), ((), ())),
                        preferred_element_type=jnp.float32)
    o_ref[0] = jnp.maximum(o + b_ref[...], 0.0)


def _tc2(gsum, W2p, b2):
    g3 = gsum.reshape(_B, _N, _C)
    return pl.pallas_call(
        _tc2_body,
        grid=(_B,),
        in_specs=[
            pl.BlockSpec((1, _N, _C), lambda b: (b, 0, 0)),
            pl.BlockSpec((_COUT, _C), lambda b: (0, 0)),
            pl.BlockSpec((_COUT, 1), lambda b: (0, 0)),
        ],
        out_specs=pl.BlockSpec((1, _COUT, _N), lambda b: (b, 0, 0)),
        out_shape=jax.ShapeDtypeStruct((_B, _COUT, _N), jnp.float32),
    )(g3, W2p, b2.reshape(_COUT, 1))


def kernel(x, hyperedge_matrix, point_hyperedge_index, centers, W1, b1, W2, b2, eps):
    t1 = _pack_cmajor(x[..., 0])
    centers_cmaj = centers[:, :, :_HE, 0]
    perm = _perm()

    boff_n = (jnp.arange(_B, dtype=jnp.int32) * (_N + 1))[:, None, None]
    idx1 = (hyperedge_matrix.astype(jnp.int32) + boff_n).reshape(_B * _HE, _KN)
    idx1t = idx1.reshape(_NW, -1, 32)

    boff_e = (jnp.arange(_B, dtype=jnp.int32) * _HE)[:, None, None]
    idx2 = (point_hyperedge_index.astype(jnp.int32) + boff_e).reshape(_B * _N, _KE)
    idx2t = idx2.reshape(_NW, -1, 32)

    hsum = _sc_gather_sum(_B * _HE, _KN, _B * (_N + 1), _C)(t1, idx1t)
    e3 = _tc1(hsum, centers_cmaj, W1[:, perm], b1, eps)
    t2 = _pack_cmajor(e3)
    gsum = _sc_gather_sum(_B * _N, _KE, _B * _HE, _C)(t2, idx2t)
    return _tc2(gsum, W2[:, perm], b2)
